# Initial kernel scaffold; baseline (speedup 1.0000x reference)
import functools

import jax
import jax.numpy as jnp
from jax import lax
from jax.experimental import pallas as pl
from jax.experimental.pallas import tpu as pltpu
from jax.experimental.pallas import tpu_sc as plsc

N_NODES = 10000
N_PAD = 10240
F = 128
N_GRAPHS = 128
NC, NS = 2, 16
NW = NC * NS
CHUNK = 128
ECHUNKS = 79
E_PAD = NW * ECHUNKS * CHUNK
RPS = N_PAD // NS

_sc_mesh = plsc.VectorSubcoreMesh(core_axis_name="c", subcore_axis_name="s")


def _sc_degree(dst_idx):
    @functools.partial(
        pl.kernel,
        out_type=jax.ShapeDtypeStruct((NC, N_PAD, 16), jnp.float32),
        mesh=_sc_mesh,
        scratch_types=[
            pltpu.VMEM((ECHUNKS, CHUNK), jnp.int32),
            pltpu.VMEM((CHUNK, 16), jnp.float32),
            pltpu.VMEM((RPS, 16), jnp.float32),
            pltpu.VMEM_SHARED((N_PAD, 16), jnp.float32),
        ],
    )
    def k(dst_hbm, out_hbm, dst_v, ones_v, zeros_v, deg_sh):
        cid = lax.axis_index("c")
        sid = lax.axis_index("s")
        wid = sid * NC + cid

        @pl.loop(0, CHUNK)
        def _(i):
            ones_v[i] = jnp.ones((16,), jnp.float32)

        @pl.loop(0, RPS)
        def _(i):
            zeros_v[i] = jnp.zeros((16,), jnp.float32)

        pltpu.sync_copy(zeros_v, deg_sh.at[pl.ds(sid * RPS, RPS)])
        pltpu.sync_copy(dst_hbm.at[wid], dst_v)
        plsc.subcore_barrier()

        @pl.loop(0, ECHUNKS)
        def _(j):
            pltpu.sync_copy(ones_v, deg_sh.at[dst_v.at[j]], add=True)

        plsc.subcore_barrier()
        pltpu.sync_copy(deg_sh.at[pl.ds(sid * RPS, RPS)],
                        out_hbm.at[cid, pl.ds(sid * RPS, RPS)])

    return k(dst_idx)


def _sc_spmm(scaled, src_idx, dst_idx):
    @functools.partial(
        pl.kernel,
        out_type=jax.ShapeDtypeStruct((NC, N_PAD, F), jnp.float32),
        mesh=_sc_mesh,
        scratch_types=[
            pltpu.VMEM((ECHUNKS, CHUNK), jnp.int32),
            pltpu.VMEM((ECHUNKS, CHUNK), jnp.int32),
            pltpu.VMEM((CHUNK, F), jnp.float32),
            pltpu.VMEM_SHARED((N_PAD, F), jnp.float32),
            pltpu.SemaphoreType.DMA,
        ],
    )
    def k(scaled_hbm, src_hbm, dst_hbm, out_hbm, src_v, dst_v, rows_v, acc_sh, sem):
        cid = lax.axis_index("c")
        sid = lax.axis_index("s")
        wid = sid * NC + cid

        pltpu.sync_copy(scaled_hbm.at[pl.ds(sid * RPS, RPS)],
                        acc_sh.at[pl.ds(sid * RPS, RPS)])
        pltpu.sync_copy(src_hbm.at[wid], src_v)
        pltpu.sync_copy(dst_hbm.at[wid], dst_v)
        plsc.subcore_barrier()

        @pl.loop(0, ECHUNKS)
        def _(j):
            pltpu.async_copy(scaled_hbm.at[src_v.at[j]], rows_v, sem).wait()
            pltpu.sync_copy(rows_v, acc_sh.at[dst_v.at[j]], add=True)

        plsc.subcore_barrier()
        pltpu.sync_copy(acc_sh.at[pl.ds(sid * RPS, RPS)],
                        out_hbm.at[cid, pl.ds(sid * RPS, RPS)])

    return k(scaled, src_idx, dst_idx)


def _dis_from_parts(d_ref):
    deg = d_ref[0][:, :1] + d_ref[1][:, :1] + 1.0
    return lax.rsqrt(deg)


def _tc_matmul(x_pad, w):
    def body(x_ref, w_ref, o_ref):
        o_ref[...] = jnp.dot(x_ref[...], w_ref[...],
                             preferred_element_type=jnp.float32)

    return pl.pallas_call(
        body, out_shape=jax.ShapeDtypeStruct((N_PAD, F), jnp.float32))(x_pad, w)


def _tc_prescale(y, deg_parts):
    def body(y_ref, d_ref, o_ref):
        o_ref[...] = _dis_from_parts(d_ref) * y_ref[...]

    return pl.pallas_call(
        body, out_shape=jax.ShapeDtypeStruct((N_PAD, F), jnp.float32))(y, deg_parts)


def _tc_combine_matmul(agg, scaled, deg_parts, b, w):
    def body(a_ref, s_ref, d_ref, b_ref, w_ref, o_ref):
        dis = _dis_from_parts(d_ref)
        h = jnp.maximum(dis * (a_ref[0] + a_ref[1] - s_ref[...]) + b_ref[...], 0.0)
        rows = lax.broadcasted_iota(jnp.int32, (N_PAD, 1), 0)
        h = jnp.where(rows < N_NODES, h, 0.0)
        o_ref[...] = dis * jnp.dot(h, w_ref[...],
                                   preferred_element_type=jnp.float32)

    return pl.pallas_call(
        body, out_shape=jax.ShapeDtypeStruct((N_PAD, F), jnp.float32))(
            agg, scaled, deg_parts, b, w)


def _tc_finish(agg, scaled, deg_parts, b, batch_row, wl, bl):
    def body(a_ref, s_ref, d_ref, b_ref, g_ref, wl_ref, bl_ref, o_ref):
        dis = _dis_from_parts(d_ref)
        h = jnp.maximum(dis * (a_ref[0] + a_ref[1] - s_ref[...]) + b_ref[...], 0.0)
        rows = lax.broadcasted_iota(jnp.int32, (N_PAD, 1), 0)
        h = jnp.where(rows < N_NODES, h, 0.0)
        gid = lax.broadcasted_iota(jnp.int32, (N_GRAPHS, N_PAD), 0)
        pt = (gid == g_ref[...]).astype(jnp.float32)
        sums = jnp.dot(pt, h, preferred_element_type=jnp.float32)
        cnt = jnp.sum(pt, axis=1)[:, None]
        pooled = sums / jnp.maximum(cnt, 1.0)
        o_ref[...] = jnp.dot(pooled, wl_ref[...],
                             preferred_element_type=jnp.float32) + bl_ref[...]

    return pl.pallas_call(
        body, out_shape=jax.ShapeDtypeStruct((N_GRAPHS, F), jnp.float32))(
            agg, scaled, deg_parts, b, batch_row, wl, bl)


def kernel(x, edge_index, batch, W1, b1, W2, b2, Wl, bl):
    i32 = jnp.int32
    src = edge_index[0].astype(i32)
    dst = edge_index[1].astype(i32)
    e = src.shape[0]
    pad = jnp.full((E_PAD - e,), N_NODES, i32)
    src_p = jnp.concatenate([src, pad]).reshape(NW, ECHUNKS, CHUNK)
    dst_p = jnp.concatenate([dst, pad]).reshape(NW, ECHUNKS, CHUNK)
    x_pad = jnp.pad(x, ((0, N_PAD - N_NODES), (0, 0)))
    batch_row = jnp.pad(batch.astype(i32), (0, N_PAD - N_NODES),
                        constant_values=N_GRAPHS).reshape(1, N_PAD)

    deg_parts = _sc_degree(dst_p)
    y1 = _tc_matmul(x_pad, W1)
    scaled1 = _tc_prescale(y1, deg_parts)
    agg1 = _sc_spmm(scaled1, src_p, dst_p)
    scaled2 = _tc_combine_matmul(agg1, scaled1, deg_parts,
                                 b1.reshape(1, F), W2)
    agg2 = _sc_spmm(scaled2, src_p, dst_p)
    return _tc_finish(agg2, scaled2, deg_parts, b2.reshape(1, F),
                      batch_row, Wl, bl.reshape(1, F))


# speedup vs baseline: 8.2488x; 8.2488x over previous
import dataclasses
import functools

import jax
import jax.numpy as jnp
from jax import lax
from jax.experimental import pallas as pl
from jax.experimental.pallas import tpu as pltpu
from jax.experimental.pallas import tpu_sc as plsc

N_NODES = 10000
F = 128
N_GRAPHS = 128
NC, NS = 2, 16
NW = NC * NS
CHUNK = 128
CORE_ROWS = 5248
N_PAD = NC * CORE_ROWS
BUF_ROWS = 5376
SCHUNKS = 158
E_PAD = NS * SCHUNKS * CHUNK
DCHUNKS = 79
HALF = 5120
INIT_RPS = CORE_ROWS // NS
OUT_RPS = BUF_ROWS // NS

_sc_mesh = plsc.VectorSubcoreMesh(core_axis_name="c", subcore_axis_name="s")

_sc_params = pltpu.CompilerParams()
if "needs_layout_passes" in pltpu.CompilerParams.__dataclass_fields__:
    _sc_params = dataclasses.replace(_sc_params, needs_layout_passes=False)


HROWS = HALF // 8


def _sc_degree(dst_idx, zeros):
    @functools.partial(
        pl.kernel,
        out_type=jax.ShapeDtypeStruct((NC, NS, HROWS, CHUNK), jnp.int32),
        mesh=_sc_mesh,
        compiler_params=_sc_params,
        scratch_types=[
            pltpu.VMEM((DCHUNKS, CHUNK), jnp.int32),
            pltpu.VMEM((HROWS, CHUNK), jnp.int32),
        ],
    )
    def k(dst_hbm, z_hbm, out_hbm, dst_v, hist):
        cid = lax.axis_index("c")
        sid = lax.axis_index("s")
        wid = sid * NC + cid
        pltpu.sync_copy(dst_hbm.at[wid], dst_v)
        pltpu.sync_copy(z_hbm, hist)
        iota = lax.iota(jnp.int32, 16)

        @pl.loop(0, DCHUNKS)
        def _(j):
            @pl.loop(0, CHUNK, step=16)
            def _(kk):
                kk = pl.multiple_of(kk, 16)
                d = dst_v[j, pl.ds(kk, 16)]
                hi = d >= HALF
                local = d - jnp.where(hi, HALF, 0)
                val = jnp.where(hi, 1 << 16, 1)
                row = local >> 3
                lane = ((local & 7) << 4) + iota
                plsc.addupdate_scatter(hist, [row, lane], val)

        pltpu.sync_copy(hist, out_hbm.at[cid, sid])

    return k(dst_idx, zeros)


def _sc_spmm(scaled, src_idx, dst_idx):
    @functools.partial(
        pl.kernel,
        out_type=jax.ShapeDtypeStruct((NC, BUF_ROWS, F), jnp.float32),
        mesh=_sc_mesh,
        compiler_params=_sc_params,
        scratch_types=[
            pltpu.VMEM((SCHUNKS, CHUNK), jnp.int32),
            pltpu.VMEM((SCHUNKS, CHUNK), jnp.int32),
            pltpu.VMEM((SCHUNKS, CHUNK), jnp.int32),
            pltpu.VMEM((CHUNK, F), jnp.float32),
            pltpu.VMEM_SHARED((BUF_ROWS, F), jnp.float32),
            pltpu.SemaphoreType.DMA,
        ],
    )
    def k(scaled_hbm, src_hbm, dst_hbm, out_hbm, src_v, dst_v, dst_r, rows_v,
          acc_sh, sem):
        cid = lax.axis_index("c")
        sid = lax.axis_index("s")
        base = cid * CORE_ROWS
        pltpu.sync_copy(scaled_hbm.at[pl.ds(base + sid * INIT_RPS, INIT_RPS)],
                        acc_sh.at[pl.ds(sid * INIT_RPS, INIT_RPS)])
        pltpu.sync_copy(src_hbm.at[sid], src_v)
        pltpu.sync_copy(dst_hbm.at[sid], dst_v)

        @pl.loop(0, SCHUNKS)
        def _(j):
            @pl.loop(0, CHUNK, step=16)
            def _(kk):
                kk = pl.multiple_of(kk, 16)
                d = dst_v[j, pl.ds(kk, 16)]
                local = d - base
                msk = (local >= 0) & (local < CORE_ROWS)
                dst_r[j, pl.ds(kk, 16)] = jnp.where(msk, local, CORE_ROWS)

        plsc.subcore_barrier()

        @pl.loop(0, SCHUNKS)
        def _(j):
            pltpu.async_copy(scaled_hbm.at[src_v.at[j]], rows_v, sem).wait()
            pltpu.sync_copy(rows_v, acc_sh.at[dst_r.at[j]], add=True)

        plsc.subcore_barrier()
        pltpu.sync_copy(acc_sh.at[pl.ds(sid * OUT_RPS, OUT_RPS)],
                        out_hbm.at[cid, pl.ds(sid * OUT_RPS, OUT_RPS)])

    return k(scaled, src_idx, dst_idx)


def _tc_dis(hists):
    def body(h_ref, o_ref):
        h = h_ref[...]
        lo = jnp.sum(h & 0xFFFF, axis=(0, 1))
        hi = jnp.sum(h >> 16, axis=(0, 1))
        for p, s in enumerate((lo, hi)):
            cnt = jnp.sum(s.reshape(HROWS, 8, 16), axis=2)
            deg = cnt.reshape(HALF, 1).astype(jnp.float32) + 1.0
            o_ref[pl.ds(p * HALF, HALF), :] = lax.rsqrt(deg)
        o_ref[pl.ds(2 * HALF, N_PAD - 2 * HALF), :] = jnp.ones(
            (N_PAD - 2 * HALF, 1), jnp.float32)

    return pl.pallas_call(
        body, out_shape=jax.ShapeDtypeStruct((N_PAD, 1), jnp.float32))(hists)


def _tc_matmul(x_pad, w):
    def body(x_ref, w_ref, o_ref):
        o_ref[...] = jnp.dot(x_ref[...], w_ref[...],
                             preferred_element_type=jnp.float32)

    return pl.pallas_call(
        body, out_shape=jax.ShapeDtypeStruct((N_PAD, F), jnp.float32))(x_pad, w)


def _tc_prescale(y, dis):
    def body(y_ref, d_ref, o_ref):
        o_ref[...] = d_ref[...] * y_ref[...]

    return pl.pallas_call(
        body, out_shape=jax.ShapeDtypeStruct((N_PAD, F), jnp.float32))(y, dis)


def _agg_h(a_ref, s_ref, d_ref, b_ref):
    del s_ref
    a = jnp.concatenate(
        [a_ref[0, :CORE_ROWS, :], a_ref[1, :CORE_ROWS, :]], axis=0)
    h = jnp.maximum(d_ref[...] * a + b_ref[...], 0.0)
    rows = lax.broadcasted_iota(jnp.int32, (N_PAD, 1), 0)
    return jnp.where(rows < N_NODES, h, 0.0)


def _tc_combine_matmul(agg, scaled, dis, b, w):
    def body(a_ref, s_ref, d_ref, b_ref, w_ref, o_ref):
        h = _agg_h(a_ref, s_ref, d_ref, b_ref)
        o_ref[...] = d_ref[...] * jnp.dot(h, w_ref[...],
                                          preferred_element_type=jnp.float32)

    return pl.pallas_call(
        body, out_shape=jax.ShapeDtypeStruct((N_PAD, F), jnp.float32))(
            agg, scaled, dis, b, w)


def _tc_finish(agg, scaled, dis, b, batch_row, wl, bl):
    def body(a_ref, s_ref, d_ref, b_ref, g_ref, wl_ref, bl_ref, o_ref):
        h = _agg_h(a_ref, s_ref, d_ref, b_ref)
        gid = lax.broadcasted_iota(jnp.int32, (N_GRAPHS, N_PAD), 0)
        pt = (gid == g_ref[...]).astype(jnp.float32)
        sums = jnp.dot(pt, h, preferred_element_type=jnp.float32)
        cnt = jnp.sum(pt, axis=1)[:, None]
        pooled = sums / jnp.maximum(cnt, 1.0)
        o_ref[...] = jnp.dot(pooled, wl_ref[...],
                             preferred_element_type=jnp.float32) + bl_ref[...]

    return pl.pallas_call(
        body, out_shape=jax.ShapeDtypeStruct((N_GRAPHS, F), jnp.float32))(
            agg, scaled, dis, b, batch_row, wl, bl)


def kernel(x, edge_index, batch, W1, b1, W2, b2, Wl, bl):
    i32 = jnp.int32
    src = edge_index[0].astype(i32)
    dst = edge_index[1].astype(i32)
    e = src.shape[0]
    pad = jnp.full((E_PAD - e,), N_NODES, i32)
    src_flat = jnp.concatenate([src, pad])
    dst_flat = jnp.concatenate([dst, pad])
    src16 = src_flat.reshape(NS, SCHUNKS, CHUNK)
    dst16 = dst_flat.reshape(NS, SCHUNKS, CHUNK)
    dst32 = dst_flat.reshape(NW, DCHUNKS, CHUNK)
    x_pad = jnp.pad(x, ((0, N_PAD - N_NODES), (0, 0)))
    batch_row = jnp.pad(batch.astype(i32), (0, N_PAD - N_NODES),
                        constant_values=N_GRAPHS).reshape(1, N_PAD)

    zeros = jnp.zeros((HROWS, CHUNK), i32)
    hists = _sc_degree(dst32, zeros)
    y1 = _tc_matmul(x_pad, W1)
    dis = _tc_dis(hists)
    scaled1 = _tc_prescale(y1, dis)
    agg1 = _sc_spmm(scaled1, src16, dst16)
    scaled2 = _tc_combine_matmul(agg1, scaled1, dis, b1.reshape(1, F), W2)
    agg2 = _sc_spmm(scaled2, src16, dst16)
    return _tc_finish(agg2, scaled2, dis, b2.reshape(1, F),
                      batch_row, Wl, bl.reshape(1, F))
